# baseline (speedup 1.0000x reference)
import functools

import jax
import jax.numpy as jnp
from jax import lax
from jax.experimental import pallas as pl
from jax.experimental.pallas import tpu as pltpu
from jax.experimental.pallas import tpu_sc as plsc

N = 10000
E = 320000
D_NODE = 128
D_EDGE = 16
EMB = 128
L = 5
B = 128
HID = 256
T = 8

NC = 2
NS = 16
NW = NC * NS
LANES = 16
EPW = E // NW
C = 40
NCHK = EPW // C
RING = 4
DL = RING - 2
NPAD = 10240
SN = NPAD // NS

_P = jax.lax.Precision.HIGHEST


def _dot(a, b):
    return jnp.dot(a.astype(jnp.bfloat16), b.astype(jnp.bfloat16),
                   preferred_element_type=jnp.float32)


def _dot_exact(a, b):
    return jnp.dot(a, b, precision=_P, preferred_element_type=jnp.float32)


def _proj(x, W, b, block_rows):
    M, K = x.shape
    _, O = W.shape
    b2 = b.reshape(1, O)

    def body(x_ref, w_ref, b_ref, o_ref):
        o_ref[...] = _dot(x_ref[...], w_ref[...]) + b_ref[...]

    return pl.pallas_call(
        body,
        grid=(M // block_rows,),
        in_specs=[
            pl.BlockSpec((block_rows, K), lambda i: (i, 0)),
            pl.BlockSpec((K, O), lambda i: (0, 0)),
            pl.BlockSpec((1, O), lambda i: (0, 0)),
        ],
        out_specs=pl.BlockSpec((block_rows, O), lambda i: (i, 0)),
        out_shape=jax.ShapeDtypeStruct((M, O), jnp.float32),
    )(x, W, b2)


@jax.jit
def _sc_edge_phase(h, e, edge_index):
    mesh = plsc.VectorSubcoreMesh(core_axis_name="c", subcore_axis_name="s")

    @functools.partial(
        pl.kernel,
        mesh=mesh,
        out_type=jax.ShapeDtypeStruct((NC * NPAD, EMB), jnp.float32),
        scratch_types=[
            pltpu.VMEM((RING, C), jnp.int32),
            pltpu.VMEM((RING, C), jnp.int32),
            pltpu.VMEM((RING, C, EMB), jnp.float32),
            pltpu.VMEM((RING, C, EMB), jnp.float32),
            pltpu.VMEM_SHARED((NPAD, EMB), jnp.float32),
            pltpu.SemaphoreType.DMA((RING,)),
            pltpu.SemaphoreType.DMA((RING,)),
            pltpu.SemaphoreType.DMA((RING,)),
        ],
    )
    def k(h_hbm, e_hbm, src_hbm, dst_hbm, out_hbm,
          src_v, dst_v, rows_v, e_v, agg_sh,
          sem_ld, sem_g, sem_s):
        cid = lax.axis_index("c")
        sid = lax.axis_index("s")
        wid = sid * NC + cid
        ebase = wid * EPW

        @pl.loop(0, C)
        def _(i):
            for j in range(EMB // LANES):
                rows_v[0, i, pl.ds(j * LANES, LANES)] = jnp.zeros(
                    (LANES,), jnp.float32)

        @pl.loop(0, SN // C)
        def _(r):
            pltpu.sync_copy(rows_v.at[0],
                            agg_sh.at[pl.ds(sid * SN + r * C, C)])
        plsc.subcore_barrier()

        def issue_loads(u, b):
            base = ebase + u * C
            pltpu.async_copy(src_hbm.at[pl.ds(base, C)], src_v.at[b],
                             sem_ld.at[b])
            pltpu.async_copy(dst_hbm.at[pl.ds(base, C)], dst_v.at[b],
                             sem_ld.at[b])
            pltpu.async_copy(e_hbm.at[pl.ds(base, C)], e_v.at[b],
                             sem_ld.at[b])

        def wait_loads(u, b):
            base = ebase + u * C
            pltpu.make_async_copy(src_hbm.at[pl.ds(base, C)], src_v.at[b],
                                  sem_ld.at[b]).wait()
            pltpu.make_async_copy(dst_hbm.at[pl.ds(base, C)], dst_v.at[b],
                                  sem_ld.at[b]).wait()
            pltpu.make_async_copy(e_hbm.at[pl.ds(base, C)], e_v.at[b],
                                  sem_ld.at[b]).wait()

        def issue_gather(b):
            pltpu.async_copy(h_hbm.at[src_v.at[b]], rows_v.at[b],
                             sem_g.at[b])

        def wait_gather(b):
            pltpu.make_async_copy(h_hbm.at[src_v.at[b]], rows_v.at[b],
                                  sem_g.at[b]).wait()

        def issue_scatter(b):
            pltpu.async_copy(rows_v.at[b], agg_sh.at[dst_v.at[b]],
                             sem_s.at[b], add=True)

        def wait_scatter(b):
            pltpu.make_async_copy(rows_v.at[b], agg_sh.at[dst_v.at[b]],
                                  sem_s.at[b]).wait()

        def compute(j):
            @plsc.parallel_loop(0, C, unroll=8)
            def _(i):
                for jj in range(EMB // LANES):
                    sl = pl.ds(jj * LANES, LANES)
                    rows_v[j, i, sl] = jnp.maximum(
                        rows_v[j, i, sl] + e_v[j, i, sl], 0.0)

        for u in range(DL):
            issue_loads(u, u)
        wait_loads(0, 0)
        issue_gather(0)

        @pl.loop(0, (NCHK - 2) // RING)
        def _(p):
            for j in range(RING):
                t = p * RING + j
                b1 = (j + 1) % RING
                bl = (j + DL) % RING
                bs = (j + RING - 2) % RING

                @pl.when(t >= 2)
                def _():
                    wait_scatter(bs)

                @pl.when(t <= NCHK - 1 - DL)
                def _():
                    issue_loads(t + DL, bl)

                wait_loads(t + 1, b1)
                issue_gather(b1)
                wait_gather(j)
                compute(j)
                issue_scatter(j)

        wait_scatter((RING - 2) % RING)
        wait_loads(NCHK - 1, 1)
        issue_gather(1)
        wait_gather(0)
        compute(0)
        issue_scatter(0)

        wait_scatter((RING - 1) % RING)
        wait_gather(1)
        compute(1)
        issue_scatter(1)

        wait_scatter(0)
        wait_scatter(1)

        plsc.subcore_barrier()
        pltpu.sync_copy(
            agg_sh.at[pl.ds(sid * SN, SN)],
            out_hbm.at[pl.ds(cid * NPAD + sid * SN, SN)])

    return k(h, e, edge_index[0], edge_index[1])


def _gin_mlp(h, p0, p1, W1l, b1l, W2l, b2l, eps1, relu_out, block_rows):
    def body(h_ref, p0_ref, p1_ref, w1_ref, b1_ref, w2_ref, b2_ref, e_ref,
             o_ref):
        z = h_ref[...] * e_ref[0, 0] + p0_ref[...] + p1_ref[...]
        a = jnp.maximum(_dot(z, w1_ref[...]) + b1_ref[...], 0.0)
        o = _dot(a, w2_ref[...]) + b2_ref[...]
        if relu_out:
            o = jnp.maximum(o, 0.0)
        o_ref[...] = o

    nb = N // block_rows
    return pl.pallas_call(
        body,
        grid=(nb,),
        in_specs=[
            pl.BlockSpec((block_rows, EMB), lambda i: (i, 0)),
            pl.BlockSpec((block_rows, EMB), lambda i: (i, 0)),
            pl.BlockSpec((block_rows, EMB), lambda i: (i, 0)),
            pl.BlockSpec((EMB, EMB), lambda i: (0, 0)),
            pl.BlockSpec((1, EMB), lambda i: (0, 0)),
            pl.BlockSpec((EMB, EMB), lambda i: (0, 0)),
            pl.BlockSpec((1, EMB), lambda i: (0, 0)),
            pl.BlockSpec((1, 1), lambda i: (0, 0)),
        ],
        out_specs=pl.BlockSpec((block_rows, EMB), lambda i: (i, 0)),
        out_shape=jax.ShapeDtypeStruct((N, EMB), jnp.float32),
    )(h, p0, p1, W1l, b1l.reshape(1, EMB), W2l, b2l.reshape(1, EMB), eps1)


def _readout_heads(h, ids_row, Wh1, bh1, Wh2, bh2, block_rows):
    nb = N // block_rows

    def body(ids_ref, h_ref, w1_ref, b1_ref, w2_ref, b2_ref, o_ref,
             sums_ref, cnts_ref):
        i = pl.program_id(0)

        @pl.when(i == 0)
        def _():
            sums_ref[...] = jnp.zeros_like(sums_ref)
            cnts_ref[...] = jnp.zeros_like(cnts_ref)

        gid = jax.lax.broadcasted_iota(jnp.int32, (B, block_rows), 0)
        mask = (ids_ref[0] == gid).astype(jnp.float32)
        sums_ref[...] += _dot_exact(mask, h_ref[...])
        cnts_ref[...] += jnp.sum(mask, axis=1, keepdims=True)

        @pl.when(i == nb - 1)
        def _():
            g = sums_ref[...] / jnp.maximum(cnts_ref[...], 1.0)
            cols = []
            for t in range(T):
                a = jnp.maximum(_dot(g, w1_ref[t]) + b1_ref[t], 0.0)
                cols.append(_dot(a, w2_ref[t]) + b2_ref[t])
            o_ref[...] = jnp.concatenate(cols, axis=1)

    return pl.pallas_call(
        body,
        grid=(nb,),
        in_specs=[
            pl.BlockSpec((1, 1, block_rows), lambda i: (i, 0, 0)),
            pl.BlockSpec((block_rows, EMB), lambda i: (i, 0)),
            pl.BlockSpec((T, EMB, HID), lambda i: (0, 0, 0)),
            pl.BlockSpec((T, 1, HID), lambda i: (0, 0, 0)),
            pl.BlockSpec((T, HID, 1), lambda i: (0, 0, 0)),
            pl.BlockSpec((T, 1, 1), lambda i: (0, 0, 0)),
        ],
        out_specs=pl.BlockSpec((B, T), lambda i: (0, 0)),
        out_shape=jax.ShapeDtypeStruct((B, T), jnp.float32),
        scratch_shapes=[
            pltpu.VMEM((B, EMB), jnp.float32),
            pltpu.VMEM((B, 1), jnp.float32),
        ],
    )(ids_row, h, Wh1, bh1.reshape(T, 1, HID), Wh2, bh2.reshape(T, 1, 1))


def kernel(node_feats, edge_feats, edge_index, node_graph_ids,
           W_node, b_node, W_edge, b_edge,
           W1, b1, W2, b2, eps,
           Wh1, bh1, Wh2, bh2):
    h = _proj(node_feats, W_node, b_node, block_rows=2000)
    e = _proj(edge_feats, W_edge, b_edge, block_rows=8000)

    for l in range(L):
        parts = _sc_edge_phase(h, e, edge_index)
        p0 = parts[:N]
        p1 = parts[NPAD:NPAD + N]
        eps1 = (1.0 + eps[l]).reshape(1, 1)
        h = _gin_mlp(h, p0, p1, W1[l], b1[l], W2[l], b2[l], eps1,
                     relu_out=(l < L - 1), block_rows=2000)

    ids_row = node_graph_ids.astype(jnp.int32).reshape(N // 2000, 1, 2000)
    return _readout_heads(h, ids_row, Wh1, bh1, Wh2, bh2, block_rows=2000)


# speedup vs baseline: 1.0231x; 1.0231x over previous
import functools

import jax
import jax.numpy as jnp
from jax import lax
from jax.experimental import pallas as pl
from jax.experimental.pallas import tpu as pltpu
from jax.experimental.pallas import tpu_sc as plsc

N = 10000
E = 320000
D_NODE = 128
D_EDGE = 16
EMB = 128
L = 5
B = 128
HID = 256
T = 8

NC = 2
NS = 16
NW = NC * NS
LANES = 16
EPW = E // NW
C = 40
NCHK = EPW // C
RING = 4
DL = RING - 2
NPAD = 10240
SN = NPAD // NS

_P = jax.lax.Precision.HIGHEST


def _dot(a, b):
    return jnp.dot(a.astype(jnp.bfloat16), b.astype(jnp.bfloat16),
                   preferred_element_type=jnp.float32)


def _dot_exact(a, b):
    return jnp.dot(a, b, precision=_P, preferred_element_type=jnp.float32)


def _proj(x, W, b, block_rows):
    M, K = x.shape
    _, O = W.shape
    b2 = b.reshape(1, O)

    def body(x_ref, w_ref, b_ref, o_ref):
        o_ref[...] = _dot(x_ref[...], w_ref[...]) + b_ref[...]

    return pl.pallas_call(
        body,
        grid=(M // block_rows,),
        in_specs=[
            pl.BlockSpec((block_rows, K), lambda i: (i, 0)),
            pl.BlockSpec((K, O), lambda i: (0, 0)),
            pl.BlockSpec((1, O), lambda i: (0, 0)),
        ],
        out_specs=pl.BlockSpec((block_rows, O), lambda i: (i, 0)),
        out_shape=jax.ShapeDtypeStruct((M, O), jnp.float32),
    )(x, W, b2)


@jax.jit
def _sc_edge_phase(h, e, edge_index):
    mesh = plsc.VectorSubcoreMesh(core_axis_name="c", subcore_axis_name="s")

    @functools.partial(
        pl.kernel,
        mesh=mesh,
        out_type=jax.ShapeDtypeStruct((NC * NPAD, EMB), jnp.float32),
        scratch_types=[
            pltpu.VMEM((RING, C), jnp.int32),
            pltpu.VMEM((RING, C), jnp.int32),
            pltpu.VMEM((RING, C, EMB), jnp.float32),
            pltpu.VMEM((RING, C, EMB), jnp.float32),
            pltpu.VMEM_SHARED((NPAD, EMB), jnp.float32),
            pltpu.SemaphoreType.DMA((RING,)),
            pltpu.SemaphoreType.DMA((RING,)),
            pltpu.SemaphoreType.DMA((RING,)),
        ],
    )
    def k(h_hbm, e_hbm, src_hbm, dst_hbm, out_hbm,
          src_v, dst_v, rows_v, e_v, agg_sh,
          sem_ld, sem_g, sem_s):
        cid = lax.axis_index("c")
        sid = lax.axis_index("s")
        wid = sid * NC + cid
        ebase = wid * EPW

        @pl.loop(0, C)
        def _(i):
            for j in range(EMB // LANES):
                rows_v[0, i, pl.ds(j * LANES, LANES)] = jnp.zeros(
                    (LANES,), jnp.float32)

        @pl.loop(0, SN // C)
        def _(r):
            pltpu.sync_copy(rows_v.at[0],
                            agg_sh.at[pl.ds(sid * SN + r * C, C)])
        plsc.subcore_barrier()

        def issue_loads(u, b):
            base = ebase + u * C
            pltpu.async_copy(src_hbm.at[pl.ds(base, C)], src_v.at[b],
                             sem_ld.at[b])
            pltpu.async_copy(dst_hbm.at[pl.ds(base, C)], dst_v.at[b],
                             sem_ld.at[b])
            pltpu.async_copy(e_hbm.at[pl.ds(base, C)], e_v.at[b],
                             sem_ld.at[b])

        def wait_loads(u, b):
            base = ebase + u * C
            pltpu.make_async_copy(src_hbm.at[pl.ds(base, C)], src_v.at[b],
                                  sem_ld.at[b]).wait()
            pltpu.make_async_copy(dst_hbm.at[pl.ds(base, C)], dst_v.at[b],
                                  sem_ld.at[b]).wait()
            pltpu.make_async_copy(e_hbm.at[pl.ds(base, C)], e_v.at[b],
                                  sem_ld.at[b]).wait()

        def issue_gather(b):
            pltpu.async_copy(h_hbm.at[src_v.at[b]], rows_v.at[b],
                             sem_g.at[b])

        def wait_gather(b):
            pltpu.make_async_copy(h_hbm.at[src_v.at[b]], rows_v.at[b],
                                  sem_g.at[b]).wait()

        def issue_scatter(b):
            pltpu.async_copy(rows_v.at[b], agg_sh.at[dst_v.at[b]],
                             sem_s.at[b], add=True)

        def wait_scatter(b):
            pltpu.make_async_copy(rows_v.at[b], agg_sh.at[dst_v.at[b]],
                                  sem_s.at[b]).wait()

        def compute(j):
            @plsc.parallel_loop(0, C, unroll=4)
            def _(i):
                for jj in range(EMB // LANES):
                    sl = pl.ds(jj * LANES, LANES)
                    rows_v[j, i, sl] = jnp.maximum(
                        rows_v[j, i, sl] + e_v[j, i, sl], 0.0)

        for u in range(DL):
            issue_loads(u, u)
        wait_loads(0, 0)
        issue_gather(0)

        @pl.loop(0, (NCHK - 2) // RING)
        def _(p):
            for j in range(RING):
                t = p * RING + j
                b1 = (j + 1) % RING
                bl = (j + DL) % RING
                bs = (j + RING - 2) % RING

                @pl.when(t >= 2)
                def _():
                    wait_scatter(bs)

                @pl.when(t <= NCHK - 1 - DL)
                def _():
                    issue_loads(t + DL, bl)

                wait_loads(t + 1, b1)
                issue_gather(b1)
                wait_gather(j)
                compute(j)
                issue_scatter(j)

        wait_scatter((RING - 2) % RING)
        wait_loads(NCHK - 1, 1)
        issue_gather(1)
        wait_gather(0)
        compute(0)
        issue_scatter(0)

        wait_scatter((RING - 1) % RING)
        wait_gather(1)
        compute(1)
        issue_scatter(1)

        wait_scatter(0)
        wait_scatter(1)

        plsc.subcore_barrier()
        pltpu.sync_copy(
            agg_sh.at[pl.ds(sid * SN, SN)],
            out_hbm.at[pl.ds(cid * NPAD + sid * SN, SN)])

    return k(h, e, edge_index[0], edge_index[1])


def _gin_mlp(h, p0, p1, W1l, b1l, W2l, b2l, eps1, relu_out, block_rows):
    def body(h_ref, p0_ref, p1_ref, w1_ref, b1_ref, w2_ref, b2_ref, e_ref,
             o_ref):
        z = h_ref[...] * e_ref[0, 0] + p0_ref[...] + p1_ref[...]
        a = jnp.maximum(_dot(z, w1_ref[...]) + b1_ref[...], 0.0)
        o = _dot(a, w2_ref[...]) + b2_ref[...]
        if relu_out:
            o = jnp.maximum(o, 0.0)
        o_ref[...] = o

    nb = N // block_rows
    return pl.pallas_call(
        body,
        grid=(nb,),
        in_specs=[
            pl.BlockSpec((block_rows, EMB), lambda i: (i, 0)),
            pl.BlockSpec((block_rows, EMB), lambda i: (i, 0)),
            pl.BlockSpec((block_rows, EMB), lambda i: (i, 0)),
            pl.BlockSpec((EMB, EMB), lambda i: (0, 0)),
            pl.BlockSpec((1, EMB), lambda i: (0, 0)),
            pl.BlockSpec((EMB, EMB), lambda i: (0, 0)),
            pl.BlockSpec((1, EMB), lambda i: (0, 0)),
            pl.BlockSpec((1, 1), lambda i: (0, 0)),
        ],
        out_specs=pl.BlockSpec((block_rows, EMB), lambda i: (i, 0)),
        out_shape=jax.ShapeDtypeStruct((N, EMB), jnp.float32),
    )(h, p0, p1, W1l, b1l.reshape(1, EMB), W2l, b2l.reshape(1, EMB), eps1)


def _readout_heads(h, ids_row, Wh1, bh1, Wh2, bh2, block_rows):
    nb = N // block_rows

    def body(ids_ref, h_ref, w1_ref, b1_ref, w2_ref, b2_ref, o_ref,
             sums_ref, cnts_ref):
        i = pl.program_id(0)

        @pl.when(i == 0)
        def _():
            sums_ref[...] = jnp.zeros_like(sums_ref)
            cnts_ref[...] = jnp.zeros_like(cnts_ref)

        gid = jax.lax.broadcasted_iota(jnp.int32, (B, block_rows), 0)
        mask = (ids_ref[0] == gid).astype(jnp.float32)
        sums_ref[...] += _dot_exact(mask, h_ref[...])
        cnts_ref[...] += jnp.sum(mask, axis=1, keepdims=True)

        @pl.when(i == nb - 1)
        def _():
            g = sums_ref[...] / jnp.maximum(cnts_ref[...], 1.0)
            cols = []
            for t in range(T):
                a = jnp.maximum(_dot(g, w1_ref[t]) + b1_ref[t], 0.0)
                cols.append(_dot(a, w2_ref[t]) + b2_ref[t])
            o_ref[...] = jnp.concatenate(cols, axis=1)

    return pl.pallas_call(
        body,
        grid=(nb,),
        in_specs=[
            pl.BlockSpec((1, 1, block_rows), lambda i: (i, 0, 0)),
            pl.BlockSpec((block_rows, EMB), lambda i: (i, 0)),
            pl.BlockSpec((T, EMB, HID), lambda i: (0, 0, 0)),
            pl.BlockSpec((T, 1, HID), lambda i: (0, 0, 0)),
            pl.BlockSpec((T, HID, 1), lambda i: (0, 0, 0)),
            pl.BlockSpec((T, 1, 1), lambda i: (0, 0, 0)),
        ],
        out_specs=pl.BlockSpec((B, T), lambda i: (0, 0)),
        out_shape=jax.ShapeDtypeStruct((B, T), jnp.float32),
        scratch_shapes=[
            pltpu.VMEM((B, EMB), jnp.float32),
            pltpu.VMEM((B, 1), jnp.float32),
        ],
    )(ids_row, h, Wh1, bh1.reshape(T, 1, HID), Wh2, bh2.reshape(T, 1, 1))


def kernel(node_feats, edge_feats, edge_index, node_graph_ids,
           W_node, b_node, W_edge, b_edge,
           W1, b1, W2, b2, eps,
           Wh1, bh1, Wh2, bh2):
    h = _proj(node_feats, W_node, b_node, block_rows=2000)
    e = _proj(edge_feats, W_edge, b_edge, block_rows=8000)

    for l in range(L):
        parts = _sc_edge_phase(h, e, edge_index)
        p0 = parts[:N]
        p1 = parts[NPAD:NPAD + N]
        eps1 = (1.0 + eps[l]).reshape(1, 1)
        h = _gin_mlp(h, p0, p1, W1[l], b1[l], W2[l], b2[l], eps1,
                     relu_out=(l < L - 1), block_rows=2000)

    ids_row = node_graph_ids.astype(jnp.int32).reshape(N // 2000, 1, 2000)
    return _readout_heads(h, ids_row, Wh1, bh1, Wh2, bh2, block_rows=2000)
